# Initial kernel scaffold; baseline (speedup 1.0000x reference)
import jax
import jax.numpy as jnp
from jax.experimental import pallas as pl


def kernel(xyz, points, conv_w0, conv_b0, bn_g0, bn_b0, conv_w1, conv_b1, bn_g1, bn_b1):
    raise NotImplementedError("write your pallas kernel here")


# speedup vs baseline: 17.0329x; 17.0329x over previous
import functools

import jax
import jax.numpy as jnp
from jax import lax
from jax.experimental import pallas as pl
from jax.experimental.pallas import tpu as pltpu
from jax.experimental.pallas import tpu_sc as plsc

B = 4
N = 8192
S = 2048
K = 16
D = 64
C1 = 64
C2 = 64
NROW = 64
NCOL = 128
TS = 128
NTOT = float(B * S * K)


def _fps_kernel(xyz_ref, newc_ref):
    xs = xyz_ref[0]
    ys = xyz_ref[1]
    zs = xyz_ref[2]
    rowi = lax.broadcasted_iota(jnp.int32, (B, NROW, NCOL), 1)
    coli = lax.broadcasted_iota(jnp.int32, (B, NROW, NCOL), 2)
    lin = rowi * NCOL + coli

    def body(i, carry):
        dist, far = carry
        sel = lin == far
        cx = jnp.sum(jnp.sum(jnp.where(sel, xs, 0.0), axis=2, keepdims=True),
                     axis=1, keepdims=True)
        cy = jnp.sum(jnp.sum(jnp.where(sel, ys, 0.0), axis=2, keepdims=True),
                     axis=1, keepdims=True)
        cz = jnp.sum(jnp.sum(jnp.where(sel, zs, 0.0), axis=2, keepdims=True),
                     axis=1, keepdims=True)
        newc_ref[0, pl.ds(i, 1), :] = cx[:, 0, 0][None, :]
        newc_ref[1, pl.ds(i, 1), :] = cy[:, 0, 0][None, :]
        newc_ref[2, pl.ds(i, 1), :] = cz[:, 0, 0][None, :]
        dx = xs - cx
        dy = ys - cy
        dz = zs - cz
        d = dx * dx + dy * dy + dz * dz
        dist = jnp.minimum(dist, d)
        m = jnp.max(jnp.max(dist, axis=2, keepdims=True), axis=1,
                    keepdims=True)
        far = jnp.min(jnp.min(jnp.where(dist == m, lin, jnp.int32(1 << 30)),
                              axis=2, keepdims=True), axis=1, keepdims=True)
        return dist, far

    dist0 = jnp.full((B, NROW, NCOL), 1e10, jnp.float32)
    far0 = jnp.zeros((B, 1, 1), jnp.int32)
    lax.fori_loop(0, S, body, (dist0, far0))


def _run_fps(xyz):
    xyzr = xyz.transpose(2, 0, 1).reshape(3, B, NROW, NCOL)
    newc = pl.pallas_call(
        _fps_kernel,
        out_shape=jax.ShapeDtypeStruct((3, S, B), jnp.float32),
    )(xyzr)
    return newc


def _knn_kernel(nc_ref, xyz_ref, idx_ref):
    b = pl.program_id(0)
    src = nc_ref[0]
    dst = xyz_ref[0]
    sx = src[:, 0:1]
    sy = src[:, 1:2]
    sz = src[:, 2:3]
    dxv = dst[0][None, :]
    dyv = dst[1][None, :]
    dzv = dst[2][None, :]
    dot = jnp.dot(src.astype(jnp.bfloat16), dst.astype(jnp.bfloat16),
                  preferred_element_type=jnp.float32)
    s2 = sx * sx + sy * sy + sz * sz
    d2 = dxv * dxv + dyv * dyv + dzv * dzv
    dists = (s2 + d2) - 2.0 * dot
    bits = lax.bitcast_convert_type(dists, jnp.int32)
    keys = jnp.where(bits >= 0, bits, jnp.int32(-2147483648) - bits)
    iota = lax.broadcasted_iota(jnp.int32, (TS, N), 1)
    base = b * N
    cols = []
    for _ in range(K):
        vmin = jnp.min(keys, axis=1, keepdims=True)
        sel = jnp.min(jnp.where(keys == vmin, iota, jnp.int32(1 << 30)),
                      axis=1, keepdims=True)
        cols.append(sel + base)
        keys = jnp.where(iota == sel, jnp.int32(2147483647), keys)
    idx_ref[0] = jnp.concatenate(cols, axis=1)


def _run_knn(new_xyz, xyz):
    xyzt = xyz.transpose(0, 2, 1)
    return pl.pallas_call(
        _knn_kernel,
        grid=(B, S // TS),
        in_specs=[
            pl.BlockSpec((1, TS, 3), lambda b, t: (b, t, 0)),
            pl.BlockSpec((1, 3, N), lambda b, t: (b, 0, 0)),
        ],
        out_specs=pl.BlockSpec((1, TS, K), lambda b, t: (b, t, 0)),
        out_shape=jax.ShapeDtypeStruct((B, S, K), jnp.int32),
    )(new_xyz, xyzt)


def _table_kernel(u_ref, w_ref, t_ref):
    t_ref[0] = jnp.dot(u_ref[0].astype(jnp.bfloat16),
                       w_ref[...].astype(jnp.bfloat16),
                       preferred_element_type=jnp.float32)


def _run_table(xyz, points, w0t):
    u = jnp.concatenate([xyz, points], axis=-1)
    return pl.pallas_call(
        _table_kernel,
        grid=(B,),
        in_specs=[
            pl.BlockSpec((1, N, 3 + D), lambda b: (b, 0, 0)),
            pl.BlockSpec((3 + D, C1), lambda b: (0, 0)),
        ],
        out_specs=pl.BlockSpec((1, N, C1), lambda b: (b, 0, 0)),
        out_shape=jax.ShapeDtypeStruct((B, N, C1), jnp.float32),
    )(u, w0t)


def _corr_kernel(nx_ref, w_ref, b0_ref, c_ref):
    c_ref[0] = jnp.dot(nx_ref[0].astype(jnp.bfloat16),
                       w_ref[...].astype(jnp.bfloat16),
                       preferred_element_type=jnp.float32) - b0_ref[...]


def _run_corr(new_xyz, w0xt, b0):
    return pl.pallas_call(
        _corr_kernel,
        grid=(B,),
        in_specs=[
            pl.BlockSpec((1, S, 3), lambda b: (b, 0, 0)),
            pl.BlockSpec((3, C1), lambda b: (0, 0)),
            pl.BlockSpec((1, C1), lambda b: (0, 0)),
        ],
        out_specs=pl.BlockSpec((1, S, C1), lambda b: (b, 0, 0)),
        out_shape=jax.ShapeDtypeStruct((B, S, C1), jnp.float32),
    )(new_xyz, w0xt, b0)


_NW = 32
_CH = 128
_ROWS = B * S * K
_PER_W = _ROWS // _NW


def _sc_gather_body(table_hbm, idx_hbm, out_hbm, idx_v, rows_v, sem):
    wid = lax.axis_index("s") * 2 + lax.axis_index("c")
    base = wid * _PER_W

    def chunk(j, carry):
        off = base + j * _CH
        pltpu.sync_copy(idx_hbm.at[pl.ds(off, _CH)], idx_v)
        pltpu.async_copy(table_hbm.at[idx_v], rows_v, sem).wait()
        pltpu.sync_copy(rows_v, out_hbm.at[pl.ds(off, _CH)])
        return carry

    lax.fori_loop(0, _PER_W // _CH, chunk, 0)


def _run_gather(table, idx_flat):
    mesh = plsc.VectorSubcoreMesh(core_axis_name="c", subcore_axis_name="s")
    fn = functools.partial(
        pl.kernel,
        mesh=mesh,
        out_type=jax.ShapeDtypeStruct((_ROWS, C1), jnp.float32),
        scratch_types=[
            pltpu.VMEM((_CH,), jnp.int32),
            pltpu.VMEM((_CH, C1), jnp.float32),
            pltpu.SemaphoreType.DMA,
        ],
        compiler_params=pltpu.CompilerParams(use_tc_tiling_on_sc=False),
    )(_sc_gather_body)
    return fn(table.reshape(B * N, C1), idx_flat)


def _bcast_corr(c):
    return jnp.broadcast_to(c[:, None, :], (TS, K, C1)).reshape(TS * K, C1)


def _stats0_kernel(g_ref, corr_ref, acc_ref):
    t = pl.program_id(0)
    y0 = g_ref[0] - _bcast_corr(corr_ref[0])
    s1 = jnp.sum(y0, axis=0)
    s2 = jnp.sum(y0 * y0, axis=0)

    @pl.when(t == 0)
    def _():
        acc_ref[...] = jnp.zeros_like(acc_ref)

    acc_ref[0, :] += s1
    acc_ref[1, :] += s2


def _scale_shift0(acc_ref, g0_ref, be0_ref):
    mean0 = acc_ref[0, :] * (1.0 / NTOT)
    var0 = acc_ref[1, :] * (1.0 / NTOT) - mean0 * mean0
    scale0 = g0_ref[0] / jnp.sqrt(var0 + 1e-5)
    shift0 = be0_ref[0] - mean0 * scale0
    return scale0, shift0


def _stats1_kernel(g_ref, corr_ref, acc_ref, g0_ref, be0_ref, m1_ref, m2_ref):
    t = pl.program_id(0)
    scale0, shift0 = _scale_shift0(acc_ref, g0_ref, be0_ref)
    y0 = g_ref[0] - _bcast_corr(corr_ref[0])
    z = jnp.maximum(y0 * scale0[None, :] + shift0[None, :], 0.0)

    @pl.when(t == 0)
    def _():
        m1_ref[...] = jnp.zeros_like(m1_ref)
        m2_ref[...] = jnp.zeros_like(m2_ref)

    m1_ref[0, :] += jnp.sum(z, axis=0)
    m2_ref[...] += lax.dot_general(z, z, (((0,), (0,)), ((), ())),
                                   preferred_element_type=jnp.float32)


def _final_kernel(g_ref, corr_ref, acc_ref, g0_ref, be0_ref, w1t_ref,
                  m1_ref, m2_ref, b1_ref, g1_ref, be1_ref, out_ref):
    scale0, shift0 = _scale_shift0(acc_ref, g0_ref, be0_ref)
    y0 = g_ref[0] - _bcast_corr(corr_ref[0])
    z = jnp.maximum(y0 * scale0[None, :] + shift0[None, :], 0.0)
    w1t = w1t_ref[...]
    y1 = jnp.dot(z.astype(jnp.bfloat16), w1t.astype(jnp.bfloat16),
                 preferred_element_type=jnp.float32) + b1_ref[...]
    r = jnp.dot(m1_ref[0:1, :], w1t, preferred_element_type=jnp.float32)[0]
    a = jnp.dot(m2_ref[...], w1t, preferred_element_type=jnp.float32)
    q = jnp.sum(a * w1t, axis=0)
    b1 = b1_ref[0]
    mean1 = r * (1.0 / NTOT) + b1
    ey2 = q * (1.0 / NTOT) + 2.0 * b1 * (r * (1.0 / NTOT)) + b1 * b1
    var1 = ey2 - mean1 * mean1
    scale1 = g1_ref[0] / jnp.sqrt(var1 + 1e-5)
    shift1 = be1_ref[0] - mean1 * scale1
    z1 = jnp.maximum(y1 * scale1[None, :] + shift1[None, :], 0.0)
    out_ref[0] = jnp.max(z1.reshape(TS, K, C2), axis=1)


def _tile_map(t):
    return (t // (S // TS), t % (S // TS), 0)


def _small(t):
    return lambda *_: tuple(0 for _ in range(t))


def _run_mlp(g, corr, g0, be0, w1t, b1, g1, be1):
    grid = (B * S // TS,)
    g_spec = pl.BlockSpec((1, TS * K, C1), _tile_map)
    c_spec = pl.BlockSpec((1, TS, C1), _tile_map)
    v_spec = pl.BlockSpec((1, C1), _small(2))

    acc = pl.pallas_call(
        _stats0_kernel,
        grid=grid,
        in_specs=[g_spec, c_spec],
        out_specs=pl.BlockSpec((8, C1), _small(2)),
        out_shape=jax.ShapeDtypeStruct((8, C1), jnp.float32),
    )(g, corr)

    m1, m2 = pl.pallas_call(
        _stats1_kernel,
        grid=grid,
        in_specs=[g_spec, c_spec, pl.BlockSpec((8, C1), _small(2)),
                  v_spec, v_spec],
        out_specs=[pl.BlockSpec((8, C1), _small(2)),
                   pl.BlockSpec((C1, C1), _small(2))],
        out_shape=[jax.ShapeDtypeStruct((8, C1), jnp.float32),
                   jax.ShapeDtypeStruct((C1, C1), jnp.float32)],
    )(g, corr, acc, g0, be0)

    out = pl.pallas_call(
        _final_kernel,
        grid=grid,
        in_specs=[g_spec, c_spec, pl.BlockSpec((8, C1), _small(2)),
                  v_spec, v_spec, pl.BlockSpec((C1, C2), _small(2)),
                  pl.BlockSpec((8, C1), _small(2)),
                  pl.BlockSpec((C1, C1), _small(2)),
                  v_spec, v_spec, v_spec],
        out_specs=pl.BlockSpec((1, TS, C2), _tile_map),
        out_shape=jax.ShapeDtypeStruct((B, S, C2), jnp.float32),
    )(g, corr, acc, g0, be0, w1t, m1, m2, b1, g1, be1)
    return out


def kernel(xyz, points, conv_w0, conv_b0, bn_g0, bn_b0,
           conv_w1, conv_b1, bn_g1, bn_b1):
    newc = _run_fps(xyz)
    new_xyz = newc.transpose(2, 1, 0)
    idx = _run_knn(new_xyz, xyz)
    w0t = conv_w0.T
    table = _run_table(xyz, points, w0t)
    corr = _run_corr(new_xyz, conv_w0[:, :3].T, conv_b0.reshape(1, C1))
    g = _run_gather(table, idx.reshape(_ROWS))
    g = g.reshape(B, S * K, C1)
    out = _run_mlp(g, corr,
                   bn_g0.reshape(1, C1), bn_b0.reshape(1, C1),
                   conv_w1.T, conv_b1.reshape(1, C2),
                   bn_g1.reshape(1, C2), bn_b1.reshape(1, C2))
    return (new_xyz, out)


# speedup vs baseline: 17.3088x; 1.0162x over previous
import functools

import jax
import jax.numpy as jnp
from jax import lax
from jax.experimental import pallas as pl
from jax.experimental.pallas import tpu as pltpu
from jax.experimental.pallas import tpu_sc as plsc

B = 4
N = 8192
S = 2048
K = 16
D = 64
C1 = 64
C2 = 64
NROW = 64
NCOL = 128
TS = 256
TS5 = 128
NTOT = float(B * S * K)


def _fps_kernel(xyz_ref, newc_ref):
    xs = xyz_ref[0]
    ys = xyz_ref[1]
    zs = xyz_ref[2]
    rowi = lax.broadcasted_iota(jnp.int32, (B, NROW, NCOL), 1)
    coli = lax.broadcasted_iota(jnp.int32, (B, NROW, NCOL), 2)
    lin = rowi * NCOL + coli

    xyz3 = xyz_ref[...]

    def body(i, carry):
        dist, far = carry
        sel = lin == far
        c3 = jnp.sum(jnp.sum(jnp.where(sel[None], xyz3, 0.0),
                             axis=3, keepdims=True),
                     axis=2, keepdims=True)
        cx = c3[0]
        cy = c3[1]
        cz = c3[2]
        newc_ref[0, pl.ds(i, 1), :] = cx[:, 0, 0][None, :]
        newc_ref[1, pl.ds(i, 1), :] = cy[:, 0, 0][None, :]
        newc_ref[2, pl.ds(i, 1), :] = cz[:, 0, 0][None, :]
        dx = xs - cx
        dy = ys - cy
        dz = zs - cz
        d = dx * dx + dy * dy + dz * dz
        dist = jnp.minimum(dist, d)
        m = jnp.max(jnp.max(dist, axis=2, keepdims=True), axis=1,
                    keepdims=True)
        far = jnp.min(jnp.min(jnp.where(dist == m, lin, jnp.int32(1 << 30)),
                              axis=2, keepdims=True), axis=1, keepdims=True)
        return dist, far

    dist0 = jnp.full((B, NROW, NCOL), 1e10, jnp.float32)
    far0 = jnp.zeros((B, 1, 1), jnp.int32)
    lax.fori_loop(0, S, body, (dist0, far0))


def _run_fps(xyz):
    xyzr = xyz.transpose(2, 0, 1).reshape(3, B, NROW, NCOL)
    newc = pl.pallas_call(
        _fps_kernel,
        out_shape=jax.ShapeDtypeStruct((3, S, B), jnp.float32),
    )(xyzr)
    return newc


def _knn_kernel(nc_ref, xyz_ref, idx_ref):
    b = pl.program_id(0)
    src = nc_ref[0]
    dst = xyz_ref[0]
    sx = src[:, 0:1]
    sy = src[:, 1:2]
    sz = src[:, 2:3]
    dxv = dst[0][None, :]
    dyv = dst[1][None, :]
    dzv = dst[2][None, :]
    dot = jnp.dot(src.astype(jnp.bfloat16), dst.astype(jnp.bfloat16),
                  preferred_element_type=jnp.float32)
    s2 = sx * sx + sy * sy + sz * sz
    d2 = dxv * dxv + dyv * dyv + dzv * dzv
    dists = (s2 + d2) - 2.0 * dot
    bits = lax.bitcast_convert_type(dists, jnp.int32)
    keys = jnp.where(bits >= 0, bits, jnp.int32(-2147483648) - bits)
    base = b * N

    lane_iota = lax.broadcasted_iota(jnp.int32, (TS, 128), 1)
    ncols = N // 128
    maxi = jnp.int32(2147483647)
    big = jnp.int32(1 << 30)
    sel_shift = None
    outs = []
    for k in range(K):
        bestv = None
        bestc = None
        newcols = []
        for c in range(ncols):
            kc = keys[:, c * 128:(c + 1) * 128]
            if sel_shift is not None:
                kc = jnp.where(sel_shift == jnp.int32(c * 128), maxi, kc)
                if k + 1 < K:
                    newcols.append(kc)
            if bestv is None:
                bestv = kc
                bestc = jnp.zeros((TS, 128), jnp.int32)
            else:
                better = kc < bestv
                bestv = jnp.where(better, kc, bestv)
                bestc = jnp.where(better, jnp.int32(c), bestc)
        if newcols:
            keys = jnp.concatenate(newcols, axis=1)
        gidx = bestc * 128 + lane_iota
        vminb = jnp.min(bestv, axis=1, keepdims=True)
        sel = jnp.min(jnp.where(bestv == vminb, gidx, big),
                      axis=1, keepdims=True)
        sel_shift = sel - lane_iota
        outs.append(sel + base)
    idx_ref[0] = jnp.concatenate(outs, axis=1)


def _run_knn(new_xyz, xyz):
    xyzt = xyz.transpose(0, 2, 1)
    return pl.pallas_call(
        _knn_kernel,
        grid=(B, S // TS),
        in_specs=[
            pl.BlockSpec((1, TS, 3), lambda b, t: (b, t, 0)),
            pl.BlockSpec((1, 3, N), lambda b, t: (b, 0, 0)),
        ],
        out_specs=pl.BlockSpec((1, TS, K), lambda b, t: (b, t, 0)),
        out_shape=jax.ShapeDtypeStruct((B, S, K), jnp.int32),
    )(new_xyz, xyzt)


def _table_kernel(u_ref, w_ref, t_ref):
    t_ref[0] = jnp.dot(u_ref[0].astype(jnp.bfloat16),
                       w_ref[...].astype(jnp.bfloat16),
                       preferred_element_type=jnp.float32)


def _run_table(xyz, points, w0t):
    u = jnp.concatenate([xyz, points], axis=-1)
    return pl.pallas_call(
        _table_kernel,
        grid=(B,),
        in_specs=[
            pl.BlockSpec((1, N, 3 + D), lambda b: (b, 0, 0)),
            pl.BlockSpec((3 + D, C1), lambda b: (0, 0)),
        ],
        out_specs=pl.BlockSpec((1, N, C1), lambda b: (b, 0, 0)),
        out_shape=jax.ShapeDtypeStruct((B, N, C1), jnp.float32),
    )(u, w0t)


def _corr_kernel(nx_ref, w_ref, b0_ref, c_ref):
    c_ref[0] = jnp.dot(nx_ref[0].astype(jnp.bfloat16),
                       w_ref[...].astype(jnp.bfloat16),
                       preferred_element_type=jnp.float32) - b0_ref[...]


def _run_corr(new_xyz, w0xt, b0):
    return pl.pallas_call(
        _corr_kernel,
        grid=(B,),
        in_specs=[
            pl.BlockSpec((1, S, 3), lambda b: (b, 0, 0)),
            pl.BlockSpec((3, C1), lambda b: (0, 0)),
            pl.BlockSpec((1, C1), lambda b: (0, 0)),
        ],
        out_specs=pl.BlockSpec((1, S, C1), lambda b: (b, 0, 0)),
        out_shape=jax.ShapeDtypeStruct((B, S, C1), jnp.float32),
    )(new_xyz, w0xt, b0)


_NW = 32
_CH = 128
_ROWS = B * S * K
_PER_W = _ROWS // _NW


def _sc_gather_body(table_hbm, idx_hbm, out_hbm, idx_v, rows_v, sem):
    wid = lax.axis_index("s") * 2 + lax.axis_index("c")
    base = wid * _PER_W

    def chunk(j, carry):
        off = base + j * _CH
        pltpu.sync_copy(idx_hbm.at[pl.ds(off, _CH)], idx_v)
        pltpu.async_copy(table_hbm.at[idx_v], rows_v, sem).wait()
        pltpu.sync_copy(rows_v, out_hbm.at[pl.ds(off, _CH)])
        return carry

    lax.fori_loop(0, _PER_W // _CH, chunk, 0)


def _run_gather(table, idx_flat):
    mesh = plsc.VectorSubcoreMesh(core_axis_name="c", subcore_axis_name="s")
    fn = functools.partial(
        pl.kernel,
        mesh=mesh,
        out_type=jax.ShapeDtypeStruct((_ROWS, C1), jnp.float32),
        scratch_types=[
            pltpu.VMEM((_CH,), jnp.int32),
            pltpu.VMEM((_CH, C1), jnp.float32),
            pltpu.SemaphoreType.DMA,
        ],
        compiler_params=pltpu.CompilerParams(use_tc_tiling_on_sc=False),
    )(_sc_gather_body)
    return fn(table.reshape(B * N, C1), idx_flat)


def _bcast_corr(c):
    return jnp.broadcast_to(c[:, None, :], (TS5, K, C1)).reshape(TS5 * K, C1)


def _stats0_kernel(g_ref, corr_ref, acc_ref):
    t = pl.program_id(0)
    y0 = g_ref[0] - _bcast_corr(corr_ref[0])
    s1 = jnp.sum(y0, axis=0)
    s2 = jnp.sum(y0 * y0, axis=0)

    @pl.when(t == 0)
    def _():
        acc_ref[...] = jnp.zeros_like(acc_ref)

    acc_ref[0, :] += s1
    acc_ref[1, :] += s2


def _scale_shift0(acc_ref, g0_ref, be0_ref):
    mean0 = acc_ref[0, :] * (1.0 / NTOT)
    var0 = acc_ref[1, :] * (1.0 / NTOT) - mean0 * mean0
    scale0 = g0_ref[0] / jnp.sqrt(var0 + 1e-5)
    shift0 = be0_ref[0] - mean0 * scale0
    return scale0, shift0


def _stats1_kernel(g_ref, corr_ref, acc_ref, g0_ref, be0_ref, m1_ref, m2_ref):
    t = pl.program_id(0)
    scale0, shift0 = _scale_shift0(acc_ref, g0_ref, be0_ref)
    y0 = g_ref[0] - _bcast_corr(corr_ref[0])
    z = jnp.maximum(y0 * scale0[None, :] + shift0[None, :], 0.0)

    @pl.when(t == 0)
    def _():
        m1_ref[...] = jnp.zeros_like(m1_ref)
        m2_ref[...] = jnp.zeros_like(m2_ref)

    m1_ref[0, :] += jnp.sum(z, axis=0)
    m2_ref[...] += lax.dot_general(z, z, (((0,), (0,)), ((), ())),
                                   preferred_element_type=jnp.float32)


def _final_kernel(g_ref, corr_ref, acc_ref, g0_ref, be0_ref, w1t_ref,
                  m1_ref, m2_ref, b1_ref, g1_ref, be1_ref, out_ref):
    scale0, shift0 = _scale_shift0(acc_ref, g0_ref, be0_ref)
    y0 = g_ref[0] - _bcast_corr(corr_ref[0])
    z = jnp.maximum(y0 * scale0[None, :] + shift0[None, :], 0.0)
    w1t = w1t_ref[...]
    y1 = jnp.dot(z.astype(jnp.bfloat16), w1t.astype(jnp.bfloat16),
                 preferred_element_type=jnp.float32) + b1_ref[...]
    r = jnp.dot(m1_ref[0:1, :], w1t, preferred_element_type=jnp.float32)[0]
    a = jnp.dot(m2_ref[...], w1t, preferred_element_type=jnp.float32)
    q = jnp.sum(a * w1t, axis=0)
    b1 = b1_ref[0]
    mean1 = r * (1.0 / NTOT) + b1
    ey2 = q * (1.0 / NTOT) + 2.0 * b1 * (r * (1.0 / NTOT)) + b1 * b1
    var1 = ey2 - mean1 * mean1
    scale1 = g1_ref[0] / jnp.sqrt(var1 + 1e-5)
    shift1 = be1_ref[0] - mean1 * scale1
    z1 = jnp.maximum(y1 * scale1[None, :] + shift1[None, :], 0.0)
    out_ref[0] = jnp.max(z1.reshape(TS5, K, C2), axis=1)


def _tile_map(t):
    return (t // (S // TS5), t % (S // TS5), 0)


def _small(t):
    return lambda *_: tuple(0 for _ in range(t))


def _run_mlp(g, corr, g0, be0, w1t, b1, g1, be1):
    grid = (B * S // TS5,)
    g_spec = pl.BlockSpec((1, TS5 * K, C1), _tile_map)
    c_spec = pl.BlockSpec((1, TS5, C1), _tile_map)
    v_spec = pl.BlockSpec((1, C1), _small(2))

    acc = pl.pallas_call(
        _stats0_kernel,
        grid=grid,
        in_specs=[g_spec, c_spec],
        out_specs=pl.BlockSpec((8, C1), _small(2)),
        out_shape=jax.ShapeDtypeStruct((8, C1), jnp.float32),
    )(g, corr)

    m1, m2 = pl.pallas_call(
        _stats1_kernel,
        grid=grid,
        in_specs=[g_spec, c_spec, pl.BlockSpec((8, C1), _small(2)),
                  v_spec, v_spec],
        out_specs=[pl.BlockSpec((8, C1), _small(2)),
                   pl.BlockSpec((C1, C1), _small(2))],
        out_shape=[jax.ShapeDtypeStruct((8, C1), jnp.float32),
                   jax.ShapeDtypeStruct((C1, C1), jnp.float32)],
    )(g, corr, acc, g0, be0)

    out = pl.pallas_call(
        _final_kernel,
        grid=grid,
        in_specs=[g_spec, c_spec, pl.BlockSpec((8, C1), _small(2)),
                  v_spec, v_spec, pl.BlockSpec((C1, C2), _small(2)),
                  pl.BlockSpec((8, C1), _small(2)),
                  pl.BlockSpec((C1, C1), _small(2)),
                  v_spec, v_spec, v_spec],
        out_specs=pl.BlockSpec((1, TS5, C2), _tile_map),
        out_shape=jax.ShapeDtypeStruct((B, S, C2), jnp.float32),
    )(g, corr, acc, g0, be0, w1t, m1, m2, b1, g1, be1)
    return out


def kernel(xyz, points, conv_w0, conv_b0, bn_g0, bn_b0,
           conv_w1, conv_b1, bn_g1, bn_b1):
    newc = _run_fps(xyz)
    new_xyz = newc.transpose(2, 1, 0)
    idx = _run_knn(new_xyz, xyz)
    w0t = conv_w0.T
    table = _run_table(xyz, points, w0t)
    corr = _run_corr(new_xyz, conv_w0[:, :3].T, conv_b0.reshape(1, C1))
    g = _run_gather(table, idx.reshape(_ROWS))
    g = g.reshape(B, S * K, C1)
    out = _run_mlp(g, corr,
                   bn_g0.reshape(1, C1), bn_b0.reshape(1, C1),
                   conv_w1.T, conv_b1.reshape(1, C2),
                   bn_g1.reshape(1, C2), bn_b1.reshape(1, C2))
    return (new_xyz, out)
